# baseline (speedup 1.0000x reference)
import functools

import jax
import jax.numpy as jnp
from jax.experimental import pallas as pl
from jax.experimental.pallas import tpu as pltpu

_K = 64
_IGNORE = -1
_BLOCK_N = 4096


def _bce_body(n_total, pred_ref, tgt_ref, out_ref, acc_ref):
    i = pl.program_id(0)
    nblk = pl.num_programs(0)

    @pl.when(i == 0)
    def _init():
        acc_ref[0] = 0.0
        acc_ref[1] = 0.0

    x = pred_ref[...]
    t = tgt_ref[...]
    b = x.shape[1]
    col = i * b + jax.lax.broadcasted_iota(jnp.int32, (1, b), 1)
    valid = (t != _IGNORE) & (col < n_total)
    vf = valid.astype(jnp.float32)

    sp = jnp.maximum(x, 0.0) + jnp.log1p(jnp.exp(-jnp.abs(x)))
    rows = jax.lax.broadcasted_iota(jnp.int32, (x.shape[0], b), 0)
    onehot = rows == t
    contrib = sp - jnp.where(onehot, x, 0.0)
    block_sum = jnp.sum(jnp.sum(contrib, axis=0, keepdims=True) * vf)
    acc_ref[0] += block_sum
    acc_ref[1] += jnp.sum(vf)

    @pl.when(i == nblk - 1)
    def _fin():
        total = acc_ref[0]
        nv = acc_ref[1]
        denom = jnp.float32(x.shape[0]) * nv
        out_ref[0, 0] = jnp.where(denom > 0.0, total / jnp.maximum(denom, 1.0), 0.0)


def kernel(pred, target):
    k, n = pred.shape
    t2 = target.astype(jnp.int32).reshape(1, n)
    grid = pl.cdiv(n, _BLOCK_N)
    out = pl.pallas_call(
        functools.partial(_bce_body, n),
        grid=(grid,),
        in_specs=[
            pl.BlockSpec((k, _BLOCK_N), lambda i: (0, i)),
            pl.BlockSpec((1, _BLOCK_N), lambda i: (0, i)),
        ],
        out_specs=pl.BlockSpec(memory_space=pltpu.SMEM),
        out_shape=jax.ShapeDtypeStruct((1, 1), jnp.float32),
        scratch_shapes=[pltpu.SMEM((2,), jnp.float32)],
    )(pred, t2)
    return out[0, 0]


# speedup vs baseline: 1.2789x; 1.2789x over previous
import functools

import jax
import jax.numpy as jnp
from jax.experimental import pallas as pl
from jax.experimental.pallas import tpu as pltpu

_K = 64
_IGNORE = -1
_BLOCK_N = 4096
_LOG2E = 1.4426950408889634
_LN2 = 0.6931471805599453


def _bce_body(n_total, pred_ref, tgt_ref, out_ref, acc_ref, nv_ref, sc_ref):
    i = pl.program_id(0)
    nblk = pl.num_programs(0)

    @pl.when(i == 0)
    def _init():
        acc_ref[...] = jnp.zeros_like(acc_ref)
        nv_ref[...] = jnp.zeros_like(nv_ref)

    x = pred_ref[...]
    t = tgt_ref[...]
    kk, b = x.shape
    col = i * b + jax.lax.broadcasted_iota(jnp.int32, (1, b), 1)
    valid = (t != _IGNORE) & (col < n_total)

    u = jnp.abs(x)
    e = jnp.exp2(-_LOG2E * u)
    lg = jnp.log2(1.0 + e)
    rows = jax.lax.broadcasted_iota(jnp.int32, (kk, b), 0)
    g = jnp.where(rows == t, x, 0.0)

    half = jnp.full((1, kk), 0.5, dtype=jnp.float32)
    ln2w = jnp.full((1, kk), _LN2, dtype=jnp.float32)
    onesw = jnp.full((1, kk), 1.0, dtype=jnp.float32)
    row = (
        jax.lax.dot(half, x + u, preferred_element_type=jnp.float32)
        + jax.lax.dot(ln2w, lg, preferred_element_type=jnp.float32)
        - jax.lax.dot(onesw, g, preferred_element_type=jnp.float32)
    )
    acc_ref[...] += jnp.where(valid, row, 0.0)
    nv_ref[...] += jnp.where(valid, 1.0, 0.0)

    @pl.when(i == nblk - 1)
    def _fin():
        total = jnp.sum(acc_ref[...])
        nv = jnp.sum(nv_ref[...])
        denom = jnp.float32(kk) * nv
        sc_ref[0] = jnp.where(denom > 0.0, total / jnp.maximum(denom, 1.0), 0.0)
        out_ref[0, 0] = sc_ref[0]


def kernel(pred, target):
    k, n = pred.shape
    t2 = target.astype(jnp.int32).reshape(1, n)
    grid = pl.cdiv(n, _BLOCK_N)
    out = pl.pallas_call(
        functools.partial(_bce_body, n),
        grid=(grid,),
        in_specs=[
            pl.BlockSpec((k, _BLOCK_N), lambda i: (0, i)),
            pl.BlockSpec((1, _BLOCK_N), lambda i: (0, i)),
        ],
        out_specs=pl.BlockSpec(memory_space=pltpu.SMEM),
        out_shape=jax.ShapeDtypeStruct((1, 1), jnp.float32),
        scratch_shapes=[
            pltpu.VMEM((1, _BLOCK_N), jnp.float32),
            pltpu.VMEM((1, _BLOCK_N), jnp.float32),
            pltpu.SMEM((1,), jnp.float32),
        ],
    )(pred, t2)
    return out[0, 0]
